# baseline (speedup 1.0000x reference)
import functools

import jax
import jax.numpy as jnp
from jax import lax
from jax.experimental import pallas as pl
from jax.experimental.pallas import tpu as pltpu
from jax.experimental.pallas import tpu_sc as plsc

N_NODES = 10000
N_AGG = 10112
E_RAW = 320000
E_PAD = 327680
LAT = 128
NW = 32
EPW = E_PAD // NW
GC = 128
GN = EPW // GC
E_T = 2560
N_T = 2000


def _dot(a, b):
    return jnp.dot(a, b, preferred_element_type=jnp.float32)


def _ln(x, g, b):
    mu = jnp.mean(x, axis=-1, keepdims=True)
    var = jnp.mean((x - mu) ** 2, axis=-1, keepdims=True)
    return (x - mu) * lax.rsqrt(var + 1e-5) * g + b


def _enc_kernel(x_ref, w1, b1, w2, b2, w3, b3, w4, b4, g, bt, o_ref):
    x = jax.nn.relu(_dot(x_ref[...], w1[...]) + b1[...])
    x = jax.nn.relu(_dot(x, w2[...]) + b2[...])
    x = jax.nn.relu(_dot(x, w3[...]) + b3[...])
    x = _dot(x, w4[...]) + b4[...]
    o_ref[...] = _ln(x, g[...], bt[...])


def _bf(x):
    return x.astype(jnp.bfloat16)


def _edge_kernel(hs_ref, hd_ref, he_ref, w1a, w1b, w1c, b1, w2, b2, w3, b3,
                 w4, b4, g, bt, ne_ref, heo_ref):
    x = _dot(_bf(hs_ref[...]), w1a[...]) + _dot(_bf(hd_ref[...]), w1b[...])
    x = jax.nn.relu(x + _dot(_bf(he_ref[...]), w1c[...]) + b1[...])
    x = jax.nn.relu(_dot(_bf(x), w2[...]) + b2[...])
    x = jax.nn.relu(_dot(_bf(x), w3[...]) + b3[...])
    x = _dot(_bf(x), w4[...]) + b4[...]
    ne = _ln(x, g[...], bt[...])
    ne_ref[...] = ne
    heo_ref[...] = he_ref[...] + ne


def _node_kernel(h_ref, agg_ref, w1a, w1b, b1, w2, b2, w3, b3, w4, b4, g, bt,
                 ho_ref):
    agg = agg_ref[0] + agg_ref[1]
    x = jax.nn.relu(_dot(_bf(h_ref[...]), w1a[...]) + _dot(_bf(agg), w1b[...])
                    + b1[...])
    x = jax.nn.relu(_dot(_bf(x), w2[...]) + b2[...])
    x = jax.nn.relu(_dot(_bf(x), w3[...]) + b3[...])
    x = _dot(_bf(x), w4[...]) + b4[...]
    ho_ref[...] = h_ref[...] + _ln(x, g[...], bt[...])


def _dec_kernel(h_ref, x_ref, w1, b1, w2, b2, w3, b3, w4, b4, g, bt, o_ref):
    x = jax.nn.relu(_dot(h_ref[...], w1[...]) + b1[...])
    x = jax.nn.relu(_dot(x, w2[...]) + b2[...])
    x = jax.nn.relu(_dot(x, w3[...]) + b3[...])
    x = _dot(x, w4[...]) + b4[...]
    o_ref[...] = _ln(x, g[...], bt[...]) + x_ref[...]


def _full(shape):
    return pl.BlockSpec(shape, lambda i: (0,) * len(shape))


def _rows(n, w):
    return pl.BlockSpec((n, w), lambda i: (i, 0))


def _run_enc(x, params, rows_total, tile):
    n_in = x.shape[1]
    grid = (rows_total // tile,)
    return pl.pallas_call(
        _enc_kernel,
        grid=grid,
        in_specs=[_rows(tile, n_in)] + [_full(p.shape) for p in params],
        out_specs=_rows(tile, LAT),
        out_shape=jax.ShapeDtypeStruct((rows_total, LAT), jnp.float32),
    )(x, *params)


def _run_edge(hs, hd, he, params):
    grid = (E_PAD // E_T,)
    return pl.pallas_call(
        _edge_kernel,
        grid=grid,
        in_specs=[_rows(E_T, LAT)] * 3 + [_full(p.shape) for p in params],
        out_specs=[_rows(E_T, LAT)] * 2,
        out_shape=[jax.ShapeDtypeStruct((E_PAD, LAT), jnp.float32)] * 2,
    )(hs, hd, he, *params)


def _run_node(h, agg2, params):
    grid = (N_NODES // N_T,)
    return pl.pallas_call(
        _node_kernel,
        grid=grid,
        in_specs=[_rows(N_T, LAT),
                  pl.BlockSpec((2, N_T, LAT), lambda i: (0, i, 0))]
                 + [_full(p.shape) for p in params],
        out_specs=_rows(N_T, LAT),
        out_shape=jax.ShapeDtypeStruct((N_NODES, LAT), jnp.float32),
    )(h, agg2, *params)


def _run_dec(h, x, params):
    grid = (N_NODES // N_T,)
    n_out = x.shape[1]
    return pl.pallas_call(
        _dec_kernel,
        grid=grid,
        in_specs=[_rows(N_T, LAT), _rows(N_T, n_out)]
                 + [_full(p.shape) for p in params],
        out_specs=_rows(N_T, n_out),
        out_shape=jax.ShapeDtypeStruct((N_NODES, n_out), jnp.float32),
    )(h, x, *params)


def _sc_mesh():
    return plsc.VectorSubcoreMesh(core_axis_name="c", subcore_axis_name="s")


def _sc_gather(h, src3, dst3):
    @functools.partial(
        pl.kernel,
        mesh=_sc_mesh(),
        out_type=[jax.ShapeDtypeStruct((E_PAD, LAT), jnp.float32)] * 2,
        scratch_types=[
            pltpu.VMEM_SHARED((N_NODES, LAT), jnp.float32),
            pltpu.VMEM((2, GC), jnp.int32),
            pltpu.VMEM((2, GC, LAT), jnp.float32),
        ] + [pltpu.SemaphoreType.DMA] * 6,
    )
    def k(h_hbm, src_hbm, dst_hbm, hs_out, hd_out, hsh, idxb, buf,
          is0, is1, gs0, gs1, ws0, ws1):
        isem = (is0, is1)
        gsem = (gs0, gs1)
        wsem = (ws0, ws1)
        cid = lax.axis_index("c")
        sid = lax.axis_index("s")
        wid = sid * 2 + cid
        base = wid * EPW

        @pl.when(sid < 15)
        def _():
            rows = pl.ds(sid * 640, 640)
            pltpu.sync_copy(h_hbm.at[rows, :], hsh.at[rows, :])

        @pl.when(sid == 15)
        def _():
            rows = pl.ds(9600, 400)
            pltpu.sync_copy(h_hbm.at[rows, :], hsh.at[rows, :])

        plsc.subcore_barrier()

        def one_pass(idx_hbm, out):
            def ldidx(j, r):
                pltpu.async_copy(idx_hbm.at[wid, j], idxb.at[r], isem[r])

            def wait_idx(r):
                pltpu.make_async_copy(idx_hbm.at[wid, 0], idxb.at[r],
                                      isem[r]).wait()

            def gath(r):
                pltpu.async_copy(hsh.at[idxb.at[r]], buf.at[r], gsem[r])

            def wait_g(r):
                pltpu.make_async_copy(hsh.at[idxb.at[r]], buf.at[r],
                                      gsem[r]).wait()

            def wr(j, r):
                rows = pl.ds(base + j * GC, GC)
                pltpu.async_copy(buf.at[r], out.at[rows, :], wsem[r])

            def wait_w(r):
                rows = pl.ds(0, GC)
                pltpu.make_async_copy(buf.at[r], out.at[rows, :],
                                      wsem[r]).wait()

            ldidx(0, 0)
            ldidx(1, 1)
            wait_idx(0)
            gath(0)
            wait_idx(1)
            gath(1)

            def body(k2, _):
                j = k2 * 2
                for r in range(2):
                    wait_g(r)
                    wr(j + r, r)

                    @pl.when(k2 < GN // 2 - 1)
                    def _():
                        ldidx(j + 2 + r, r)
                for r in range(2):
                    @pl.when(k2 < GN // 2 - 1)
                    def _():
                        wait_w(r)
                        wait_idx(r)
                        gath(r)
                return ()

            lax.fori_loop(0, GN // 2, body, ())
            wait_w(0)
            wait_w(1)

        one_pass(src_hbm, hs_out)
        one_pass(dst_hbm, hd_out)

    return k(h, src3, dst3)


def _sc_scatter(ne, dst3, zseed):
    rows_per_sub = N_AGG // 16
    nbuf = 2
    niter = GN // nbuf

    @functools.partial(
        pl.kernel,
        mesh=_sc_mesh(),
        out_type=jax.ShapeDtypeStruct((2, N_AGG, LAT), jnp.float32),
        scratch_types=[
            pltpu.VMEM_SHARED((N_AGG, LAT), jnp.float32),
            pltpu.VMEM((GN, GC), jnp.int32),
            pltpu.VMEM((nbuf, GC, LAT), jnp.float32),
        ] + [pltpu.SemaphoreType.DMA] * (2 * nbuf),
    )
    def k(ne_hbm, dst_hbm, z_hbm, out, acc, didx, rbuf, *sems):
        rsem = sems[:nbuf]
        ssem = sems[nbuf:]
        cid = lax.axis_index("c")
        sid = lax.axis_index("s")
        wid = sid * 2 + cid
        base = wid * EPW
        zrows = pl.ds(sid * rows_per_sub, rows_per_sub)
        pltpu.sync_copy(z_hbm, acc.at[zrows, :])
        pltpu.sync_copy(dst_hbm.at[wid], didx)
        plsc.subcore_barrier()

        def rd(j, r):
            rows = pl.ds(base + j * GC, GC)
            pltpu.async_copy(ne_hbm.at[rows, :], rbuf.at[r], rsem[r])

        def wait_rd(r):
            rows = pl.ds(0, GC)
            pltpu.make_async_copy(ne_hbm.at[rows, :], rbuf.at[r],
                                  rsem[r]).wait()

        def scat(j, r):
            pltpu.async_copy(rbuf.at[r], acc.at[didx.at[j]], ssem[r],
                             add=True)

        def wait_scat(r):
            pltpu.make_async_copy(rbuf.at[r], acc.at[didx.at[0]],
                                  ssem[r]).wait()

        for r in range(nbuf):
            rd(r, r)

        def body(k2, _):
            j = k2 * nbuf
            for r in range(nbuf):
                wait_rd(r)
                scat(j + r, r)

            @pl.when(k2 < niter - 1)
            def _():
                for r in range(nbuf):
                    wait_scat(r)
                    rd(j + nbuf + r, r)
            return ()

        lax.fori_loop(0, niter, body, ())
        for r in range(nbuf):
            wait_scat(r)
        plsc.subcore_barrier()
        pltpu.sync_copy(acc.at[zrows, :], out.at[cid, zrows, :])

    return k(ne, dst3, zseed)


def _prep_mlp(p, in_scale=None, in_shift=None, out_scale=None, out_shift=None):
    (w1, b1), (w2, b2), (w3, b3), (w4, b4), (g, bt) = p
    if in_scale is not None:
        b1 = b1 + in_shift @ w1
        w1 = in_scale[:, None] * w1
    if out_scale is not None:
        bt = bt * out_scale + out_shift
        g = g * out_scale
    r = lambda v: v.reshape(1, -1)
    return [w1, r(b1), w2, r(b2), w3, r(b3), w4, r(b4), r(g), r(bt)]


def kernel(x, edge_attr, edge_index, enc_node_mlp, enc_edge_mlp,
           blocks_edge_mlps, blocks_node_mlps, dec_mlp, node_mean, node_std,
           edge_mean, edge_std, out_mean, out_std):
    pad = E_PAD - E_RAW
    src = jnp.concatenate([edge_index[0], jnp.zeros((pad,), jnp.int32)])
    dst_g = jnp.concatenate([edge_index[1], jnp.zeros((pad,), jnp.int32)])
    dst_s = jnp.concatenate([edge_index[1],
                             jnp.full((pad,), N_NODES, jnp.int32)])
    src3 = src.reshape(NW, GN, GC)
    dstg3 = dst_g.reshape(NW, GN, GC)
    dsts3 = dst_s.reshape(NW, GN, GC)
    ea = jnp.concatenate(
        [edge_attr, jnp.zeros((pad, edge_attr.shape[1]), jnp.float32)])
    zseed = jnp.zeros((N_AGG // 16, LAT), jnp.float32)

    enc_n = _prep_mlp(enc_node_mlp, 1.0 / node_std, -(node_mean / node_std))
    enc_e = _prep_mlp(enc_edge_mlp, 1.0 / edge_std, -(edge_mean / edge_std))
    dec = _prep_mlp(dec_mlp, out_scale=out_std, out_shift=out_mean)

    bf = jnp.bfloat16
    eblocks = []
    for p in blocks_edge_mlps:
        q = _prep_mlp(p)
        w1 = q[0].astype(bf)
        e = [w1[:LAT], w1[LAT:2 * LAT], w1[2 * LAT:]] + q[1:]
        for ix in (4, 6, 8):
            e[ix] = e[ix].astype(bf)
        eblocks.append(e)
    nblocks = []
    for p in blocks_node_mlps:
        q = _prep_mlp(p)
        w1 = q[0].astype(bf)
        nb = [w1[:LAT], w1[LAT:]] + q[1:]
        for ix in (3, 5, 7):
            nb[ix] = nb[ix].astype(bf)
        nblocks.append(nb)

    h = _run_enc(x, enc_n, N_NODES, N_T)
    he = _run_enc(ea, enc_e, E_PAD, E_T)

    for i in range(15):
        hs, hd = _sc_gather(h, src3, dstg3)
        ne, he = _run_edge(hs, hd, he, eblocks[i])
        agg2 = _sc_scatter(ne, dsts3, zseed)
        h = _run_node(h, agg2, nblocks[i])

    return _run_dec(h, x, dec)


# speedup vs baseline: 1.0381x; 1.0381x over previous
import functools

import jax
import jax.numpy as jnp
from jax import lax
from jax.experimental import pallas as pl
from jax.experimental.pallas import tpu as pltpu
from jax.experimental.pallas import tpu_sc as plsc

N_NODES = 10000
N_AGG = 10112
E_RAW = 320000
E_PAD = 327680
LAT = 128
NW = 32
EPW = E_PAD // NW
GC = 128
GN = EPW // GC
E_T = 2560
N_T = 2000


def _dot(a, b):
    return jnp.dot(a, b, preferred_element_type=jnp.float32)


def _ln(x, g, b):
    mu = jnp.mean(x, axis=-1, keepdims=True)
    var = jnp.mean((x - mu) ** 2, axis=-1, keepdims=True)
    return (x - mu) * lax.rsqrt(var + 1e-5) * g + b


def _enc_kernel(x_ref, w1, b1, w2, b2, w3, b3, w4, b4, g, bt, o_ref):
    x = jax.nn.relu(_dot(x_ref[...], w1[...]) + b1[...])
    x = jax.nn.relu(_dot(x, w2[...]) + b2[...])
    x = jax.nn.relu(_dot(x, w3[...]) + b3[...])
    x = _dot(x, w4[...]) + b4[...]
    o_ref[...] = _ln(x, g[...], bt[...]).astype(o_ref.dtype)


def _bf(x):
    return x.astype(jnp.bfloat16)


def _edge_kernel(hs_ref, hd_ref, he_ref, w1a, w1b, w1c, b1, w2, b2, w3, b3,
                 w4, b4, g, bt, ne_ref, heo_ref):
    x = _dot(_bf(hs_ref[...]), w1a[...]) + _dot(_bf(hd_ref[...]), w1b[...])
    x = jax.nn.relu(x + _dot(_bf(he_ref[...]), w1c[...]) + b1[...])
    x = jax.nn.relu(_dot(_bf(x), w2[...]) + b2[...])
    x = jax.nn.relu(_dot(_bf(x), w3[...]) + b3[...])
    x = _dot(_bf(x), w4[...]) + b4[...]
    ne = _ln(x, g[...], bt[...])
    ne_ref[...] = ne
    heo_ref[...] = (he_ref[...].astype(jnp.float32) + ne).astype(jnp.bfloat16)


def _node_kernel(h_ref, agg_ref, w1a, w1b, b1, w2, b2, w3, b3, w4, b4, g, bt,
                 ho_ref):
    agg = agg_ref[0] + agg_ref[1]
    x = jax.nn.relu(_dot(_bf(h_ref[...]), w1a[...]) + _dot(_bf(agg), w1b[...])
                    + b1[...])
    x = jax.nn.relu(_dot(_bf(x), w2[...]) + b2[...])
    x = jax.nn.relu(_dot(_bf(x), w3[...]) + b3[...])
    x = _dot(_bf(x), w4[...]) + b4[...]
    ho_ref[...] = h_ref[...] + _ln(x, g[...], bt[...])


def _dec_kernel(h_ref, x_ref, w1, b1, w2, b2, w3, b3, w4, b4, g, bt, o_ref):
    x = jax.nn.relu(_dot(h_ref[...], w1[...]) + b1[...])
    x = jax.nn.relu(_dot(x, w2[...]) + b2[...])
    x = jax.nn.relu(_dot(x, w3[...]) + b3[...])
    x = _dot(x, w4[...]) + b4[...]
    o_ref[...] = _ln(x, g[...], bt[...]) + x_ref[...]


def _full(shape):
    return pl.BlockSpec(shape, lambda i: (0,) * len(shape))


def _rows(n, w):
    return pl.BlockSpec((n, w), lambda i: (i, 0))


def _run_enc(x, params, rows_total, tile, out_dtype=jnp.float32):
    n_in = x.shape[1]
    grid = (rows_total // tile,)
    return pl.pallas_call(
        _enc_kernel,
        grid=grid,
        in_specs=[_rows(tile, n_in)] + [_full(p.shape) for p in params],
        out_specs=_rows(tile, LAT),
        out_shape=jax.ShapeDtypeStruct((rows_total, LAT), out_dtype),
    )(x, *params)


def _run_edge(hs, hd, he, params):
    grid = (E_PAD // E_T,)
    return pl.pallas_call(
        _edge_kernel,
        grid=grid,
        in_specs=[_rows(E_T, LAT)] * 3 + [_full(p.shape) for p in params],
        out_specs=[_rows(E_T, LAT)] * 2,
        out_shape=[jax.ShapeDtypeStruct((E_PAD, LAT), jnp.float32),
                   jax.ShapeDtypeStruct((E_PAD, LAT), jnp.bfloat16)],
    )(hs, hd, he, *params)


def _run_node(h, agg2, params):
    grid = (N_NODES // N_T,)
    return pl.pallas_call(
        _node_kernel,
        grid=grid,
        in_specs=[_rows(N_T, LAT),
                  pl.BlockSpec((2, N_T, LAT), lambda i: (0, i, 0))]
                 + [_full(p.shape) for p in params],
        out_specs=_rows(N_T, LAT),
        out_shape=jax.ShapeDtypeStruct((N_NODES, LAT), jnp.float32),
    )(h, agg2, *params)


def _run_dec(h, x, params):
    grid = (N_NODES // N_T,)
    n_out = x.shape[1]
    return pl.pallas_call(
        _dec_kernel,
        grid=grid,
        in_specs=[_rows(N_T, LAT), _rows(N_T, n_out)]
                 + [_full(p.shape) for p in params],
        out_specs=_rows(N_T, n_out),
        out_shape=jax.ShapeDtypeStruct((N_NODES, n_out), jnp.float32),
    )(h, x, *params)


def _sc_mesh():
    return plsc.VectorSubcoreMesh(core_axis_name="c", subcore_axis_name="s")


def _sc_gather(h, src3, dst3):
    @functools.partial(
        pl.kernel,
        mesh=_sc_mesh(),
        out_type=[jax.ShapeDtypeStruct((E_PAD, LAT), jnp.float32)] * 2,
        scratch_types=[
            pltpu.VMEM_SHARED((N_NODES, LAT), jnp.float32),
            pltpu.VMEM((2, GC), jnp.int32),
            pltpu.VMEM((2, GC, LAT), jnp.float32),
        ] + [pltpu.SemaphoreType.DMA] * 6,
    )
    def k(h_hbm, src_hbm, dst_hbm, hs_out, hd_out, hsh, idxb, buf,
          is0, is1, gs0, gs1, ws0, ws1):
        isem = (is0, is1)
        gsem = (gs0, gs1)
        wsem = (ws0, ws1)
        cid = lax.axis_index("c")
        sid = lax.axis_index("s")
        wid = sid * 2 + cid
        base = wid * EPW

        @pl.when(sid < 15)
        def _():
            rows = pl.ds(sid * 640, 640)
            pltpu.sync_copy(h_hbm.at[rows, :], hsh.at[rows, :])

        @pl.when(sid == 15)
        def _():
            rows = pl.ds(9600, 400)
            pltpu.sync_copy(h_hbm.at[rows, :], hsh.at[rows, :])

        plsc.subcore_barrier()

        def one_pass(idx_hbm, out):
            def ldidx(j, r):
                pltpu.async_copy(idx_hbm.at[wid, j], idxb.at[r], isem[r])

            def wait_idx(r):
                pltpu.make_async_copy(idx_hbm.at[wid, 0], idxb.at[r],
                                      isem[r]).wait()

            def gath(r):
                pltpu.async_copy(hsh.at[idxb.at[r]], buf.at[r], gsem[r])

            def wait_g(r):
                pltpu.make_async_copy(hsh.at[idxb.at[r]], buf.at[r],
                                      gsem[r]).wait()

            def wr(j, r):
                rows = pl.ds(base + j * GC, GC)
                pltpu.async_copy(buf.at[r], out.at[rows, :], wsem[r])

            def wait_w(r):
                rows = pl.ds(0, GC)
                pltpu.make_async_copy(buf.at[r], out.at[rows, :],
                                      wsem[r]).wait()

            ldidx(0, 0)
            ldidx(1, 1)
            wait_idx(0)
            gath(0)
            wait_idx(1)
            gath(1)

            def body(k2, _):
                j = k2 * 2
                for r in range(2):
                    wait_g(r)
                    wr(j + r, r)

                    @pl.when(k2 < GN // 2 - 1)
                    def _():
                        ldidx(j + 2 + r, r)
                for r in range(2):
                    @pl.when(k2 < GN // 2 - 1)
                    def _():
                        wait_w(r)
                        wait_idx(r)
                        gath(r)
                return ()

            lax.fori_loop(0, GN // 2, body, ())
            wait_w(0)
            wait_w(1)

        one_pass(src_hbm, hs_out)
        one_pass(dst_hbm, hd_out)

    return k(h, src3, dst3)


def _sc_scatter(ne, dst3, zseed):
    rows_per_sub = N_AGG // 16
    nbuf = 2
    niter = GN // nbuf

    @functools.partial(
        pl.kernel,
        mesh=_sc_mesh(),
        out_type=jax.ShapeDtypeStruct((2, N_AGG, LAT), jnp.float32),
        scratch_types=[
            pltpu.VMEM_SHARED((N_AGG, LAT), jnp.float32),
            pltpu.VMEM((GN, GC), jnp.int32),
            pltpu.VMEM((nbuf, GC, LAT), jnp.float32),
        ] + [pltpu.SemaphoreType.DMA] * (2 * nbuf),
    )
    def k(ne_hbm, dst_hbm, z_hbm, out, acc, didx, rbuf, *sems):
        rsem = sems[:nbuf]
        ssem = sems[nbuf:]
        cid = lax.axis_index("c")
        sid = lax.axis_index("s")
        wid = sid * 2 + cid
        base = wid * EPW
        zrows = pl.ds(sid * rows_per_sub, rows_per_sub)
        pltpu.sync_copy(z_hbm, acc.at[zrows, :])
        pltpu.sync_copy(dst_hbm.at[wid], didx)
        plsc.subcore_barrier()

        def rd(j, r):
            rows = pl.ds(base + j * GC, GC)
            pltpu.async_copy(ne_hbm.at[rows, :], rbuf.at[r], rsem[r])

        def wait_rd(r):
            rows = pl.ds(0, GC)
            pltpu.make_async_copy(ne_hbm.at[rows, :], rbuf.at[r],
                                  rsem[r]).wait()

        def scat(j, r):
            pltpu.async_copy(rbuf.at[r], acc.at[didx.at[j]], ssem[r],
                             add=True)

        def wait_scat(r):
            pltpu.make_async_copy(rbuf.at[r], acc.at[didx.at[0]],
                                  ssem[r]).wait()

        for r in range(nbuf):
            rd(r, r)

        def body(k2, _):
            j = k2 * nbuf
            for r in range(nbuf):
                wait_rd(r)
                scat(j + r, r)

            @pl.when(k2 < niter - 1)
            def _():
                for r in range(nbuf):
                    wait_scat(r)
                    rd(j + nbuf + r, r)
            return ()

        lax.fori_loop(0, niter, body, ())
        for r in range(nbuf):
            wait_scat(r)
        plsc.subcore_barrier()
        pltpu.sync_copy(acc.at[zrows, :], out.at[cid, zrows, :])

    return k(ne, dst3, zseed)


def _prep_mlp(p, in_scale=None, in_shift=None, out_scale=None, out_shift=None):
    (w1, b1), (w2, b2), (w3, b3), (w4, b4), (g, bt) = p
    if in_scale is not None:
        b1 = b1 + in_shift @ w1
        w1 = in_scale[:, None] * w1
    if out_scale is not None:
        bt = bt * out_scale + out_shift
        g = g * out_scale
    r = lambda v: v.reshape(1, -1)
    return [w1, r(b1), w2, r(b2), w3, r(b3), w4, r(b4), r(g), r(bt)]


def kernel(x, edge_attr, edge_index, enc_node_mlp, enc_edge_mlp,
           blocks_edge_mlps, blocks_node_mlps, dec_mlp, node_mean, node_std,
           edge_mean, edge_std, out_mean, out_std):
    pad = E_PAD - E_RAW
    src = jnp.concatenate([edge_index[0], jnp.zeros((pad,), jnp.int32)])
    dst_g = jnp.concatenate([edge_index[1], jnp.zeros((pad,), jnp.int32)])
    dst_s = jnp.concatenate([edge_index[1],
                             jnp.full((pad,), N_NODES, jnp.int32)])
    src3 = src.reshape(NW, GN, GC)
    dstg3 = dst_g.reshape(NW, GN, GC)
    dsts3 = dst_s.reshape(NW, GN, GC)
    ea = jnp.concatenate(
        [edge_attr, jnp.zeros((pad, edge_attr.shape[1]), jnp.float32)])
    zseed = jnp.zeros((N_AGG // 16, LAT), jnp.float32)

    enc_n = _prep_mlp(enc_node_mlp, 1.0 / node_std, -(node_mean / node_std))
    enc_e = _prep_mlp(enc_edge_mlp, 1.0 / edge_std, -(edge_mean / edge_std))
    dec = _prep_mlp(dec_mlp, out_scale=out_std, out_shift=out_mean)

    bf = jnp.bfloat16
    eblocks = []
    for p in blocks_edge_mlps:
        q = _prep_mlp(p)
        w1 = q[0].astype(bf)
        e = [w1[:LAT], w1[LAT:2 * LAT], w1[2 * LAT:]] + q[1:]
        for ix in (4, 6, 8):
            e[ix] = e[ix].astype(bf)
        eblocks.append(e)
    nblocks = []
    for p in blocks_node_mlps:
        q = _prep_mlp(p)
        w1 = q[0].astype(bf)
        nb = [w1[:LAT], w1[LAT:]] + q[1:]
        for ix in (3, 5, 7):
            nb[ix] = nb[ix].astype(bf)
        nblocks.append(nb)

    h = _run_enc(x, enc_n, N_NODES, N_T)
    he = _run_enc(ea, enc_e, E_PAD, E_T, out_dtype=jnp.bfloat16)

    for i in range(15):
        hs, hd = _sc_gather(h, src3, dstg3)
        ne, he = _run_edge(hs, hd, he, eblocks[i])
        agg2 = _sc_scatter(ne, dsts3, zseed)
        h = _run_node(h, agg2, nblocks[i])

    return _run_dec(h, x, dec)
